# baseline (speedup 1.0000x reference)
import functools

import jax
import jax.numpy as jnp
from jax import lax
from jax.experimental import pallas as pl
from jax.experimental.pallas import tpu as pltpu
from jax.experimental.pallas import tpu_sc as plsc

N = 10000
E = 640000
G = 64
D_IN = 12
H = 32
P = 3

NW = 32
NP = N + 16
CHUNK = 32000


_SC_PARAMS = pltpu.CompilerParams(needs_layout_passes=False)


def _wid():
    return lax.axis_index("s") * 2 + lax.axis_index("c")


def _mesh():
    return plsc.VectorSubcoreMesh(core_axis_name="c", subcore_axis_name="s")


def _zero(ref, n):
    @plsc.parallel_loop(0, n, unroll=4)
    def _(i):
        ref[pl.ds(i * 16, 16)] = jnp.zeros((16,), jnp.float32)


def _deg_body(pk_hbm, out_hbm, pk_v, acc_v):
    w = _wid()
    shard = E // NW
    pltpu.sync_copy(pk_hbm.at[pl.ds(w * shard, shard)], pk_v)
    _zero(acc_v, NP // 16)
    ones = jnp.ones((16,), jnp.float32)

    @plsc.parallel_loop(0, shard // 16, unroll=8)
    def _(j):
        word = pk_v[pl.ds(j * 16, 16)]
        cidx = lax.shift_right_logical(word, 16)
        plsc.addupdate_scatter(acc_v, [cidx], ones)

    pltpu.sync_copy(acc_v, out_hbm.at[w])


def _deg_partials(packed):
    return pl.kernel(
        _deg_body,
        out_type=jax.ShapeDtypeStruct((NW, NP), jnp.float32),
        mesh=_mesh(),
        compiler_params=_SC_PARAMS,
        scratch_types=[
            pltpu.VMEM((E // NW,), jnp.int32),
            pltpu.VMEM((NP,), jnp.float32),
        ],
    )(packed)


def _agg_body(nfeat, nshard, g_hbm, pk_hbm, out_hbm,
              g_v, acc_v, pk0, pk1, sem0, sem1):
    w = _wid()
    f = w % nfeat
    s = w // nfeat

    @pl.when(s < nshard)
    def _():
        pltpu.sync_copy(g_hbm.at[f], g_v)
        _zero(acc_v, NP // 16)

        shard = E // nshard
        base = s * shard
        nch = shard // CHUNK

        def start(k, buf, sem):
            pltpu.async_copy(pk_hbm.at[pl.ds(base + k * CHUNK, CHUNK)],
                             buf, sem)

        def wait(k, buf, sem):
            pltpu.make_async_copy(pk_hbm.at[pl.ds(base + k * CHUNK, CHUNK)],
                                  buf, sem).wait()

        def inner(buf):
            @plsc.parallel_loop(0, CHUNK // 16, unroll=8)
            def _(j):
                word = buf[pl.ds(j * 16, 16)]
                r = word & 0xFFFF
                cidx = lax.shift_right_logical(word, 16)
                vals = plsc.load_gather(g_v, [r])
                plsc.addupdate_scatter(acc_v, [cidx], vals)

        start(0, pk0, sem0)

        def pair(i, c):
            k = 2 * i

            @pl.when(k + 1 < nch)
            def _():
                start(k + 1, pk1, sem1)

            wait(k, pk0, sem0)
            inner(pk0)

            @pl.when(k + 2 < nch)
            def _():
                start(k + 2, pk0, sem0)

            @pl.when(k + 1 < nch)
            def _():
                wait(k + 1, pk1, sem1)
                inner(pk1)

            return c

        lax.fori_loop(0, (nch + 1) // 2, pair, 0)
        pltpu.sync_copy(acc_v, out_hbm.at[s, f])


def _edge_agg(g, packed, nfeat, nshard):
    body = functools.partial(_agg_body, nfeat, nshard)
    return pl.kernel(
        body,
        out_type=jax.ShapeDtypeStruct((nshard, nfeat, NP), jnp.float32),
        mesh=_mesh(),
        compiler_params=_SC_PARAMS,
        scratch_types=[
            pltpu.VMEM((NP,), jnp.float32),
            pltpu.VMEM((NP,), jnp.float32),
            pltpu.VMEM((CHUNK,), jnp.int32),
            pltpu.VMEM((CHUNK,), jnp.int32),
            pltpu.SemaphoreType.DMA,
            pltpu.SemaphoreType.DMA,
        ],
    )(g, packed)


def _pair_body(gp_hbm, pk_hbm, out_hbm, gp_v, a0, a1, pk0, pk1, sem0, sem1):
    w = _wid()
    p = w % (H // 2)
    s = w // (H // 2)

    pltpu.sync_copy(gp_hbm.at[p], gp_v)
    _zero(a0, NP // 16)
    _zero(a1, NP // 16)

    shard = E // 2
    base = s * shard
    nch = shard // CHUNK

    def start(k, buf, sem):
        pltpu.async_copy(pk_hbm.at[pl.ds(base + k * CHUNK, CHUNK)], buf, sem)

    def wait(k, buf, sem):
        pltpu.make_async_copy(pk_hbm.at[pl.ds(base + k * CHUNK, CHUNK)],
                              buf, sem).wait()

    def inner(buf):
        @plsc.parallel_loop(0, CHUNK // 16, unroll=8)
        def _(j):
            word = buf[pl.ds(j * 16, 16)]
            r = word & 0xFFFF
            cidx = lax.shift_right_logical(word, 16)
            gw = plsc.load_gather(gp_v, [r])
            lo = plsc.bitcast(gw << 16, jnp.float32)
            hi = plsc.bitcast(gw & (-65536), jnp.float32)
            plsc.addupdate_scatter(a0, [cidx], lo)
            plsc.addupdate_scatter(a1, [cidx], hi)

    start(0, pk0, sem0)

    def pair(i, c):
        k = 2 * i

        @pl.when(k + 1 < nch)
        def _():
            start(k + 1, pk1, sem1)

        wait(k, pk0, sem0)
        inner(pk0)

        @pl.when(k + 2 < nch)
        def _():
            start(k + 2, pk0, sem0)

        @pl.when(k + 1 < nch)
        def _():
            wait(k + 1, pk1, sem1)
            inner(pk1)

        return c

    lax.fori_loop(0, (nch + 1) // 2, pair, 0)
    pltpu.sync_copy(a0, out_hbm.at[s, 2 * p])
    pltpu.sync_copy(a1, out_hbm.at[s, 2 * p + 1])


def _edge_agg_pairs(gp, packed):
    return pl.kernel(
        _pair_body,
        out_type=jax.ShapeDtypeStruct((2, H, NP), jnp.float32),
        mesh=_mesh(),
        compiler_params=_SC_PARAMS,
        scratch_types=[
            pltpu.VMEM((NP,), jnp.int32),
            pltpu.VMEM((NP,), jnp.float32),
            pltpu.VMEM((NP,), jnp.float32),
            pltpu.VMEM((CHUNK,), jnp.int32),
            pltpu.VMEM((CHUNK,), jnp.int32),
            pltpu.SemaphoreType.DMA,
            pltpu.SemaphoreType.DMA,
        ],
    )(gp, packed)


def _pack_pairs(g):
    gu = lax.bitcast_convert_type(g.astype(jnp.bfloat16), jnp.uint16)
    gu = gu.astype(jnp.uint32).reshape(g.shape[0] // 2, 2, g.shape[1])
    packed = (gu[:, 1, :] << 16) | gu[:, 0, :]
    return lax.bitcast_convert_type(packed, jnp.int32)


def _unpack_pairs(gp):
    lo = lax.bitcast_convert_type(gp << 16, jnp.float32)
    hi = lax.bitcast_convert_type(gp & jnp.int32(-65536), jnp.float32)
    st = jnp.concatenate([lo[:, None, :], hi[:, None, :]], axis=1)
    return st.reshape(2 * gp.shape[0], gp.shape[1])


def _tc1_body(parts_ref, xp_ref, w1_ref, dinv_ref, g1_ref):
    deg = 1.0 + jnp.sum(parts_ref[...], axis=0, keepdims=True)
    dinv = lax.rsqrt(deg)
    dinv_ref[...] = dinv
    xw = lax.dot_general(w1_ref[...], xp_ref[...],
                         (((0,), (1,)), ((), ())),
                         preferred_element_type=jnp.float32)
    g1_ref[...] = _pack_pairs(xw * dinv)


def _tc1(parts, xp, w1):
    return pl.pallas_call(
        _tc1_body,
        out_shape=(
            jax.ShapeDtypeStruct((1, NP), jnp.float32),
            jax.ShapeDtypeStruct((H // 2, NP), jnp.int32),
        ),
    )(parts, xp, w1)


def _tc_mid_body(pack_out, parts_ref, gp_ref, dinv_ref, b_ref, wm_ref, bm_ref,
                 wn_ref, out_ref):
    dinv = dinv_ref[...]
    agg = parts_ref[0] + parts_ref[1] + _unpack_pairs(gp_ref[...])
    h = jnp.maximum(agg * dinv + b_ref[...], 0.0)
    hm = lax.dot_general(wm_ref[...], h, (((0,), (0,)), ((), ())),
                         preferred_element_type=jnp.float32) + bm_ref[...]
    hm = jnp.maximum(hm, 0.0)
    gn = lax.dot_general(wn_ref[...], hm, (((0,), (0,)), ((), ())),
                         preferred_element_type=jnp.float32)
    gn = gn * dinv
    if pack_out:
        out_ref[...] = _pack_pairs(gn)
    else:
        out_ref[...] = gn


def _tc_mid(agg2, gp, dinv, b_col, wm, bm_col, wn, nf_out, pack_out):
    if pack_out:
        oshape = jax.ShapeDtypeStruct((nf_out // 2, NP), jnp.int32)
    else:
        oshape = jax.ShapeDtypeStruct((nf_out, NP), jnp.float32)
    return pl.pallas_call(
        functools.partial(_tc_mid_body, pack_out),
        out_shape=oshape,
    )(agg2, gp, dinv, b_col, wm, bm_col, wn)


def _tc3_body(parts_ref, g3_ref, dinv_ref, b3_ref, batch_ref, out_ref):
    agg = g3_ref[...]
    for s in range(parts_ref.shape[0]):
        agg = agg + parts_ref[s]
    h3 = agg * dinv_ref[...] + b3_ref[...]
    gids = lax.broadcasted_iota(jnp.int32, (G, NP), 0)
    oh = (gids == batch_ref[...]).astype(jnp.float32)
    pooled = lax.dot_general(oh, h3, (((1,), (1,)), ((), ())),
                             preferred_element_type=jnp.float32)
    m = jnp.max(pooled, axis=1, keepdims=True)
    ex = jnp.exp(pooled - m)
    lse = jnp.log(jnp.sum(ex, axis=1, keepdims=True))
    out_ref[...] = pooled - m - lse


def _tc3(parts3, g3, dinv, b3_col, batch2d):
    return pl.pallas_call(
        _tc3_body,
        out_shape=jax.ShapeDtypeStruct((G, P), jnp.float32),
    )(parts3, g3, dinv, b3_col, batch2d)


def kernel(x, edge_index, batch, W1, b1, Wm1, bm1, W2, b2, Wm2, bm2, W3, b3):
    ei = edge_index.astype(jnp.int32)
    packed = ei[1] * 65536 + ei[0]

    xp = jnp.pad(x, ((0, NP - N), (0, 0)))
    batch2d = jnp.pad(batch.astype(jnp.int32), (0, NP - N),
                      constant_values=G).reshape(1, NP)
    b1c = b1.reshape(H, 1)
    bm1c = bm1.reshape(H, 1)
    b2c = b2.reshape(H, 1)
    bm2c = bm2.reshape(H, 1)
    b3c = b3.reshape(P, 1)

    deg_parts = _deg_partials(packed)
    dinv, gp1 = _tc1(deg_parts, xp, W1)

    agg1 = _edge_agg_pairs(gp1, packed)
    gp2 = _tc_mid(agg1, gp1, dinv, b1c, Wm1, bm1c, W2, H, True)

    agg2 = _edge_agg_pairs(gp2, packed)
    g3 = _tc_mid(agg2, gp2, dinv, b2c, Wm2, bm2c, W3, P, False)

    parts3 = _edge_agg(g3, packed, P, 10)
    return _tc3(parts3, g3, dinv, b3c, batch2d)


# speedup vs baseline: 1.0183x; 1.0183x over previous
import functools

import jax
import jax.numpy as jnp
from jax import lax
from jax.experimental import pallas as pl
from jax.experimental.pallas import tpu as pltpu
from jax.experimental.pallas import tpu_sc as plsc

N = 10000
E = 640000
G = 64
D_IN = 12
H = 32
P = 3

NW = 32
NP = N + 16
CHUNK = 16000
CHUNK3 = 8000


_SC_PARAMS = pltpu.CompilerParams(needs_layout_passes=False)


def _wid():
    return lax.axis_index("s") * 2 + lax.axis_index("c")


def _mesh():
    return plsc.VectorSubcoreMesh(core_axis_name="c", subcore_axis_name="s")


def _zero(ref, n):
    @plsc.parallel_loop(0, n, unroll=4)
    def _(i):
        ref[pl.ds(i * 16, 16)] = jnp.zeros((16,), jnp.float32)


def _deg_body(pk_hbm, out_hbm, pk_v, acc_v, sem):
    w = _wid()
    shard = E // NW
    cp = pltpu.async_copy(pk_hbm.at[pl.ds(w * shard, shard)], pk_v, sem)
    _zero(acc_v, NP // 16)
    cp.wait()
    ones = jnp.ones((16,), jnp.float32)

    @plsc.parallel_loop(0, shard // 16, unroll=8)
    def _(j):
        word = pk_v[pl.ds(j * 16, 16)]
        cidx = lax.shift_right_logical(word, 16)
        plsc.addupdate_scatter(acc_v, [cidx], ones)

    pltpu.sync_copy(acc_v, out_hbm.at[w])


def _deg_partials(packed):
    return pl.kernel(
        _deg_body,
        out_type=jax.ShapeDtypeStruct((NW, NP), jnp.float32),
        mesh=_mesh(),
        compiler_params=_SC_PARAMS,
        scratch_types=[
            pltpu.VMEM((E // NW,), jnp.int32),
            pltpu.VMEM((NP,), jnp.float32),
            pltpu.SemaphoreType.DMA,
        ],
    )(packed)


def _chunk_loop(pk_hbm, base, nch, chunk, pk0, pk1, sem0, sem1, inner):
    def start(k, buf, sem):
        pltpu.async_copy(pk_hbm.at[pl.ds(base + k * chunk, chunk)], buf, sem)

    def wait(k, buf, sem):
        pltpu.make_async_copy(pk_hbm.at[pl.ds(base + k * chunk, chunk)],
                              buf, sem).wait()

    def pair(i, c):
        k = 2 * i

        @pl.when(k + 1 < nch)
        def _():
            start(k + 1, pk1, sem1)

        wait(k, pk0, sem0)
        inner(pk0)

        @pl.when(k + 2 < nch)
        def _():
            start(k + 2, pk0, sem0)

        @pl.when(k + 1 < nch)
        def _():
            wait(k + 1, pk1, sem1)
            inner(pk1)

        return c

    lax.fori_loop(0, (nch + 1) // 2, pair, 0)


def _p3_body(gp_hbm, gf_hbm, pk_hbm, out_hbm,
             gp_v, gf_v, a0, a1, pk0, pk1, sem0, sem1):
    w = _wid()
    is_pair = w < 16
    s = w % 16
    shard = E // 16
    nch = shard // CHUNK3
    base = s * shard
    pltpu.async_copy(pk_hbm.at[pl.ds(base, CHUNK3)], pk0, sem0)

    @pl.when(is_pair)
    def _():
        pltpu.sync_copy(gp_hbm, gp_v)
        _zero(a0, NP // 16)
        _zero(a1, NP // 16)

        def inner(buf):
            @plsc.parallel_loop(0, CHUNK3 // 16, unroll=8)
            def _(j):
                word = buf[pl.ds(j * 16, 16)]
                r = word & 0xFFFF
                cidx = lax.shift_right_logical(word, 16)
                gw = plsc.load_gather(gp_v, [r])
                lo = plsc.bitcast(gw << 16, jnp.float32)
                hi = plsc.bitcast(gw & (-65536), jnp.float32)
                plsc.addupdate_scatter(a0, [cidx], lo)
                plsc.addupdate_scatter(a1, [cidx], hi)

        _chunk_loop(pk_hbm, base, nch, CHUNK3, pk0, pk1, sem0, sem1, inner)
        pltpu.sync_copy(a0, out_hbm.at[3 * s])
        pltpu.sync_copy(a1, out_hbm.at[3 * s + 1])

    @pl.when(jnp.logical_not(is_pair))
    def _():
        pltpu.sync_copy(gf_hbm, gf_v)
        _zero(a0, NP // 16)

        def inner(buf):
            @plsc.parallel_loop(0, CHUNK3 // 16, unroll=8)
            def _(j):
                word = buf[pl.ds(j * 16, 16)]
                r = word & 0xFFFF
                cidx = lax.shift_right_logical(word, 16)
                vals = plsc.load_gather(gf_v, [r])
                plsc.addupdate_scatter(a0, [cidx], vals)

        _chunk_loop(pk_hbm, base, nch, CHUNK3, pk0, pk1, sem0, sem1, inner)
        pltpu.sync_copy(a0, out_hbm.at[3 * s + 2])


def _edge_agg_p3(gp3, g3f, packed):
    return pl.kernel(
        _p3_body,
        out_type=jax.ShapeDtypeStruct((16 * P, NP), jnp.float32),
        mesh=_mesh(),
        compiler_params=_SC_PARAMS,
        scratch_types=[
            pltpu.VMEM((NP,), jnp.int32),
            pltpu.VMEM((NP,), jnp.float32),
            pltpu.VMEM((NP,), jnp.float32),
            pltpu.VMEM((NP,), jnp.float32),
            pltpu.VMEM((CHUNK3,), jnp.int32),
            pltpu.VMEM((CHUNK3,), jnp.int32),
            pltpu.SemaphoreType.DMA,
            pltpu.SemaphoreType.DMA,
        ],
    )(gp3, g3f, packed)


def _pair_body(gp_hbm, pk_hbm, out_hbm, gp_v, a0, a1, pk0, pk1, sem0, sem1):
    w = _wid()
    p = w % (H // 2)
    s = w // (H // 2)

    shard = E // 2
    base = s * shard
    nch = shard // CHUNK
    pltpu.async_copy(pk_hbm.at[pl.ds(base, CHUNK)], pk0, sem0)

    pltpu.sync_copy(gp_hbm.at[p], gp_v)
    _zero(a0, NP // 16)
    _zero(a1, NP // 16)

    def inner(buf):
        @plsc.parallel_loop(0, CHUNK // 16, unroll=8)
        def _(j):
            word = buf[pl.ds(j * 16, 16)]
            r = word & 0xFFFF
            cidx = lax.shift_right_logical(word, 16)
            gw = plsc.load_gather(gp_v, [r])
            lo = plsc.bitcast(gw << 16, jnp.float32)
            hi = plsc.bitcast(gw & (-65536), jnp.float32)
            plsc.addupdate_scatter(a0, [cidx], lo)
            plsc.addupdate_scatter(a1, [cidx], hi)

    _chunk_loop(pk_hbm, base, nch, CHUNK, pk0, pk1, sem0, sem1, inner)
    pltpu.sync_copy(a0, out_hbm.at[s, 2 * p])
    pltpu.sync_copy(a1, out_hbm.at[s, 2 * p + 1])


def _edge_agg_pairs(gp, packed):
    return pl.kernel(
        _pair_body,
        out_type=jax.ShapeDtypeStruct((2, H, NP), jnp.float32),
        mesh=_mesh(),
        compiler_params=_SC_PARAMS,
        scratch_types=[
            pltpu.VMEM((NP,), jnp.int32),
            pltpu.VMEM((NP,), jnp.float32),
            pltpu.VMEM((NP,), jnp.float32),
            pltpu.VMEM((CHUNK,), jnp.int32),
            pltpu.VMEM((CHUNK,), jnp.int32),
            pltpu.SemaphoreType.DMA,
            pltpu.SemaphoreType.DMA,
        ],
    )(gp, packed)


def _pack_pairs(g):
    gu = lax.bitcast_convert_type(g.astype(jnp.bfloat16), jnp.uint16)
    gu = gu.astype(jnp.uint32).reshape(g.shape[0] // 2, 2, g.shape[1])
    packed = (gu[:, 1, :] << 16) | gu[:, 0, :]
    return lax.bitcast_convert_type(packed, jnp.int32)


def _unpack_pairs(gp):
    lo = lax.bitcast_convert_type(gp << 16, jnp.float32)
    hi = lax.bitcast_convert_type(gp & jnp.int32(-65536), jnp.float32)
    st = jnp.concatenate([lo[:, None, :], hi[:, None, :]], axis=1)
    return st.reshape(2 * gp.shape[0], gp.shape[1])


def _tc1_body(parts_ref, xp_ref, w1_ref, dinv_ref, g1_ref):
    deg = 1.0 + jnp.sum(parts_ref[...], axis=0, keepdims=True)
    dinv = lax.rsqrt(deg)
    dinv_ref[...] = dinv
    xw = lax.dot_general(w1_ref[...], xp_ref[...],
                         (((0,), (1,)), ((), ())),
                         preferred_element_type=jnp.float32)
    g1_ref[...] = _pack_pairs(xw * dinv)


def _tc1(parts, xp, w1):
    return pl.pallas_call(
        _tc1_body,
        out_shape=(
            jax.ShapeDtypeStruct((1, NP), jnp.float32),
            jax.ShapeDtypeStruct((H // 2, NP), jnp.int32),
        ),
    )(parts, xp, w1)


def _tc_mid_body(pack_out, parts_ref, gp_ref, dinv_ref, b_ref, wm_ref, bm_ref,
                 wn_ref, *out_refs):
    dinv = dinv_ref[...]
    agg = parts_ref[0] + parts_ref[1] + _unpack_pairs(gp_ref[...])
    h = jnp.maximum(agg * dinv + b_ref[...], 0.0)
    hm = lax.dot_general(wm_ref[...], h, (((0,), (0,)), ((), ())),
                         preferred_element_type=jnp.float32) + bm_ref[...]
    hm = jnp.maximum(hm, 0.0)
    gn = lax.dot_general(wn_ref[...], hm, (((0,), (0,)), ((), ())),
                         preferred_element_type=jnp.float32)
    gn = gn * dinv
    if pack_out:
        out_refs[0][...] = _pack_pairs(gn)
    else:
        lo = lax.bitcast_convert_type(gn[0:1].astype(jnp.bfloat16),
                                      jnp.uint16).astype(jnp.uint32)
        hi = lax.bitcast_convert_type(gn[1:2].astype(jnp.bfloat16),
                                      jnp.uint16).astype(jnp.uint32)
        out_refs[0][...] = lax.bitcast_convert_type((hi << 16) | lo, jnp.int32)
        out_refs[1][...] = gn[2:3]


def _tc_mid(agg2, gp, dinv, b_col, wm, bm_col, wn, nf_out, pack_out):
    if pack_out:
        oshape = jax.ShapeDtypeStruct((nf_out // 2, NP), jnp.int32)
    else:
        oshape = (jax.ShapeDtypeStruct((1, NP), jnp.int32),
                  jax.ShapeDtypeStruct((1, NP), jnp.float32))
    return pl.pallas_call(
        functools.partial(_tc_mid_body, pack_out),
        out_shape=oshape,
    )(agg2, gp, dinv, b_col, wm, bm_col, wn)


def _tc3_body(parts_ref, gp3_ref, gf3_ref, dinv_ref, b3_ref, batch_ref,
              out_ref):
    gp3 = gp3_ref[...]
    lo3 = lax.bitcast_convert_type(gp3 << 16, jnp.float32)
    hi3 = lax.bitcast_convert_type(gp3 & jnp.int32(-65536), jnp.float32)
    agg = jnp.concatenate([lo3, hi3, gf3_ref[...]], axis=0)
    for s in range(parts_ref.shape[0]):
        agg = agg + parts_ref[s]
    h3 = agg * dinv_ref[...] + b3_ref[...]
    gids = lax.broadcasted_iota(jnp.int32, (G, NP), 0)
    oh = (gids == batch_ref[...]).astype(jnp.float32)
    pooled = lax.dot_general(oh, h3, (((1,), (1,)), ((), ())),
                             preferred_element_type=jnp.float32)
    m = jnp.max(pooled, axis=1, keepdims=True)
    ex = jnp.exp(pooled - m)
    lse = jnp.log(jnp.sum(ex, axis=1, keepdims=True))
    out_ref[...] = pooled - m - lse


def _tc3(parts3, gp3, g3f, dinv, b3_col, batch2d):
    return pl.pallas_call(
        _tc3_body,
        out_shape=jax.ShapeDtypeStruct((G, P), jnp.float32),
    )(parts3, gp3, g3f, dinv, b3_col, batch2d)


def kernel(x, edge_index, batch, W1, b1, Wm1, bm1, W2, b2, Wm2, bm2, W3, b3):
    ei = edge_index.astype(jnp.int32)
    packed = ei[1] * 65536 + ei[0]

    xp = jnp.pad(x, ((0, NP - N), (0, 0)))
    batch2d = jnp.pad(batch.astype(jnp.int32), (0, NP - N),
                      constant_values=G).reshape(1, NP)
    b1c = b1.reshape(H, 1)
    bm1c = bm1.reshape(H, 1)
    b2c = b2.reshape(H, 1)
    bm2c = bm2.reshape(H, 1)
    b3c = b3.reshape(P, 1)

    deg_parts = _deg_partials(packed)
    dinv, gp1 = _tc1(deg_parts, xp, W1)

    agg1 = _edge_agg_pairs(gp1, packed)
    gp2 = _tc_mid(agg1, gp1, dinv, b1c, Wm1, bm1c, W2, H, True)

    agg2 = _edge_agg_pairs(gp2, packed)
    gp3, g3f = _tc_mid(agg2, gp2, dinv, b2c, Wm2, bm2c, W3, P, False)

    parts3 = _edge_agg_p3(gp3.reshape(NP), g3f.reshape(NP),
                          packed).reshape(16, P, NP)
    return _tc3(parts3, gp3, g3f, dinv, b3c, batch2d)


# speedup vs baseline: 1.0404x; 1.0217x over previous
import functools

import jax
import jax.numpy as jnp
from jax import lax
from jax.experimental import pallas as pl
from jax.experimental.pallas import tpu as pltpu
from jax.experimental.pallas import tpu_sc as plsc

N = 10000
E = 640000
G = 64
D_IN = 12
H = 32
P = 3

NW = 32
NP = N + 16
CHUNK = 16000
CHUNK3 = 8000


_SC_PARAMS = pltpu.CompilerParams(needs_layout_passes=False)


def _wid():
    return lax.axis_index("s") * 2 + lax.axis_index("c")


def _mesh():
    return plsc.VectorSubcoreMesh(core_axis_name="c", subcore_axis_name="s")


def _zero(ref, n):
    @plsc.parallel_loop(0, n, unroll=4)
    def _(i):
        ref[pl.ds(i * 16, 16)] = jnp.zeros((16,), jnp.float32)


def _deg_body(pk_hbm, out_hbm, pk_v, acc_v, sem):
    w = _wid()
    shard = E // NW
    cp = pltpu.async_copy(pk_hbm.at[pl.ds(w * shard, shard)], pk_v, sem)
    _zero(acc_v, NP // 16)
    cp.wait()
    ones = jnp.ones((16,), jnp.float32)

    @plsc.parallel_loop(0, shard // 16, unroll=8)
    def _(j):
        word = pk_v[pl.ds(j * 16, 16)]
        cidx = lax.shift_right_logical(word, 16)
        plsc.addupdate_scatter(acc_v, [cidx], ones)

    pltpu.sync_copy(acc_v, out_hbm.at[w])


def _deg_partials(packed):
    return pl.kernel(
        _deg_body,
        out_type=jax.ShapeDtypeStruct((NW, NP), jnp.float32),
        mesh=_mesh(),
        compiler_params=_SC_PARAMS,
        scratch_types=[
            pltpu.VMEM((E // NW,), jnp.int32),
            pltpu.VMEM((NP,), jnp.float32),
            pltpu.SemaphoreType.DMA,
        ],
    )(packed)


def _chunk_loop(pk_hbm, base, nch, chunk, pk0, pk1, sem0, sem1, inner):
    def start(k, buf, sem):
        pltpu.async_copy(pk_hbm.at[pl.ds(base + k * chunk, chunk)], buf, sem)

    def wait(k, buf, sem):
        pltpu.make_async_copy(pk_hbm.at[pl.ds(base + k * chunk, chunk)],
                              buf, sem).wait()

    def pair(i, c):
        k = 2 * i

        @pl.when(k + 1 < nch)
        def _():
            start(k + 1, pk1, sem1)

        wait(k, pk0, sem0)
        inner(pk0)

        @pl.when(k + 2 < nch)
        def _():
            start(k + 2, pk0, sem0)

        @pl.when(k + 1 < nch)
        def _():
            wait(k + 1, pk1, sem1)
            inner(pk1)

        return c

    lax.fori_loop(0, (nch + 1) // 2, pair, 0)


def _p3_body(gt_hbm, pk_hbm, out_hbm,
             gp_v, a0, a1, pk0, pk1, sem0, sem1):
    w = _wid()
    is_pair = w < 16
    s = w % 16
    shard = E // 16
    nch = shard // CHUNK3
    base = s * shard
    pltpu.async_copy(pk_hbm.at[pl.ds(base, CHUNK3)], pk0, sem0)

    @pl.when(is_pair)
    def _():
        pltpu.sync_copy(gt_hbm.at[0], gp_v)
        _zero(a0, NP // 16)
        _zero(a1, NP // 16)

        def inner(buf):
            @plsc.parallel_loop(0, CHUNK3 // 16, unroll=8)
            def _(j):
                word = buf[pl.ds(j * 16, 16)]
                r = word & 0xFFFF
                cidx = lax.shift_right_logical(word, 16)
                gw = plsc.load_gather(gp_v, [r])
                lo = plsc.bitcast(gw << 16, jnp.float32)
                hi = plsc.bitcast(gw & (-65536), jnp.float32)
                plsc.addupdate_scatter(a0, [cidx], lo)
                plsc.addupdate_scatter(a1, [cidx], hi)

        _chunk_loop(pk_hbm, base, nch, CHUNK3, pk0, pk1, sem0, sem1, inner)
        pltpu.sync_copy(a0, out_hbm.at[3 * s])
        pltpu.sync_copy(a1, out_hbm.at[3 * s + 1])

    @pl.when(jnp.logical_not(is_pair))
    def _():
        pltpu.sync_copy(gt_hbm.at[1], gp_v)
        _zero(a0, NP // 16)

        def inner(buf):
            @plsc.parallel_loop(0, CHUNK3 // 16, unroll=8)
            def _(j):
                word = buf[pl.ds(j * 16, 16)]
                r = word & 0xFFFF
                cidx = lax.shift_right_logical(word, 16)
                vals = plsc.bitcast(plsc.load_gather(gp_v, [r]), jnp.float32)
                plsc.addupdate_scatter(a0, [cidx], vals)

        _chunk_loop(pk_hbm, base, nch, CHUNK3, pk0, pk1, sem0, sem1, inner)
        pltpu.sync_copy(a0, out_hbm.at[3 * s + 2])


def _edge_agg_p3(gt3, packed):
    return pl.kernel(
        _p3_body,
        out_type=jax.ShapeDtypeStruct((16 * P, NP), jnp.float32),
        mesh=_mesh(),
        compiler_params=_SC_PARAMS,
        scratch_types=[
            pltpu.VMEM((NP,), jnp.int32),
            pltpu.VMEM((NP,), jnp.float32),
            pltpu.VMEM((NP,), jnp.float32),
            pltpu.VMEM((CHUNK3,), jnp.int32),
            pltpu.VMEM((CHUNK3,), jnp.int32),
            pltpu.SemaphoreType.DMA,
            pltpu.SemaphoreType.DMA,
        ],
    )(gt3, packed)


def _pair_body(gp_hbm, pk_hbm, out_hbm, gp_v, a0, a1, pk0, pk1, sem0, sem1):
    w = _wid()
    p = w % (H // 2)
    s = w // (H // 2)

    shard = E // 2
    base = s * shard
    nch = shard // CHUNK
    pltpu.async_copy(pk_hbm.at[pl.ds(base, CHUNK)], pk0, sem0)

    pltpu.sync_copy(gp_hbm.at[p], gp_v)
    _zero(a0, NP // 16)
    _zero(a1, NP // 16)

    def inner(buf):
        @plsc.parallel_loop(0, CHUNK // 16, unroll=8)
        def _(j):
            word = buf[pl.ds(j * 16, 16)]
            r = word & 0xFFFF
            cidx = lax.shift_right_logical(word, 16)
            gw = plsc.load_gather(gp_v, [r])
            lo = plsc.bitcast(gw << 16, jnp.float32)
            hi = plsc.bitcast(gw & (-65536), jnp.float32)
            plsc.addupdate_scatter(a0, [cidx], lo)
            plsc.addupdate_scatter(a1, [cidx], hi)

    _chunk_loop(pk_hbm, base, nch, CHUNK, pk0, pk1, sem0, sem1, inner)
    pltpu.sync_copy(a0, out_hbm.at[s, 2 * p])
    pltpu.sync_copy(a1, out_hbm.at[s, 2 * p + 1])


def _edge_agg_pairs(gp, packed):
    return pl.kernel(
        _pair_body,
        out_type=jax.ShapeDtypeStruct((2, H, NP), jnp.float32),
        mesh=_mesh(),
        compiler_params=_SC_PARAMS,
        scratch_types=[
            pltpu.VMEM((NP,), jnp.int32),
            pltpu.VMEM((NP,), jnp.float32),
            pltpu.VMEM((NP,), jnp.float32),
            pltpu.VMEM((CHUNK,), jnp.int32),
            pltpu.VMEM((CHUNK,), jnp.int32),
            pltpu.SemaphoreType.DMA,
            pltpu.SemaphoreType.DMA,
        ],
    )(gp, packed)


def _pack_pairs(g):
    gu = lax.bitcast_convert_type(g.astype(jnp.bfloat16), jnp.uint16)
    gu = gu.astype(jnp.uint32).reshape(g.shape[0] // 2, 2, g.shape[1])
    packed = (gu[:, 1, :] << 16) | gu[:, 0, :]
    return lax.bitcast_convert_type(packed, jnp.int32)


def _unpack_pairs(gp):
    lo = lax.bitcast_convert_type(gp << 16, jnp.float32)
    hi = lax.bitcast_convert_type(gp & jnp.int32(-65536), jnp.float32)
    st = jnp.concatenate([lo[:, None, :], hi[:, None, :]], axis=1)
    return st.reshape(2 * gp.shape[0], gp.shape[1])


def _tc1_body(parts_ref, xp_ref, w1_ref, dinv_ref, g1_ref):
    deg = 1.0 + jnp.sum(parts_ref[...], axis=0, keepdims=True)
    dinv = lax.rsqrt(deg)
    dinv_ref[...] = dinv
    xw = lax.dot_general(w1_ref[...], xp_ref[...],
                         (((0,), (1,)), ((), ())),
                         preferred_element_type=jnp.float32)
    g1_ref[...] = _pack_pairs(xw * dinv)


def _tc1(parts, xp, w1):
    return pl.pallas_call(
        _tc1_body,
        out_shape=(
            jax.ShapeDtypeStruct((1, NP), jnp.float32),
            jax.ShapeDtypeStruct((H // 2, NP), jnp.int32),
        ),
    )(parts, xp, w1)


def _tc_mid_body(pack_out, parts_ref, gp_ref, dinv_ref, b_ref, wm_ref, bm_ref,
                 wn_ref, *out_refs):
    dinv = dinv_ref[...]
    agg = parts_ref[0] + parts_ref[1] + _unpack_pairs(gp_ref[...])
    h = jnp.maximum(agg * dinv + b_ref[...], 0.0)
    hm = lax.dot_general(wm_ref[...], h, (((0,), (0,)), ((), ())),
                         preferred_element_type=jnp.float32) + bm_ref[...]
    hm = jnp.maximum(hm, 0.0)
    gn = lax.dot_general(wn_ref[...], hm, (((0,), (0,)), ((), ())),
                         preferred_element_type=jnp.float32)
    gn = gn * dinv
    if pack_out:
        out_refs[0][...] = _pack_pairs(gn)
    else:
        lo = lax.bitcast_convert_type(gn[0:1].astype(jnp.bfloat16),
                                      jnp.uint16).astype(jnp.uint32)
        hi = lax.bitcast_convert_type(gn[1:2].astype(jnp.bfloat16),
                                      jnp.uint16).astype(jnp.uint32)
        pair_row = lax.bitcast_convert_type((hi << 16) | lo, jnp.int32)
        f2_row = lax.bitcast_convert_type(gn[2:3], jnp.int32)
        zero_rows = jnp.zeros((6, gn.shape[1]), jnp.int32)
        out_refs[0][...] = jnp.concatenate([pair_row, f2_row, zero_rows], 0)


def _tc_mid(agg2, gp, dinv, b_col, wm, bm_col, wn, nf_out, pack_out):
    if pack_out:
        oshape = jax.ShapeDtypeStruct((nf_out // 2, NP), jnp.int32)
    else:
        oshape = jax.ShapeDtypeStruct((8, NP), jnp.int32)
    return pl.pallas_call(
        functools.partial(_tc_mid_body, pack_out),
        out_shape=oshape,
    )(agg2, gp, dinv, b_col, wm, bm_col, wn)


def _tc3_body(parts_ref, gt3_ref, dinv_ref, b3_ref, batch_ref, out_ref):
    gp3 = gt3_ref[0:1]
    lo3 = lax.bitcast_convert_type(gp3 << 16, jnp.float32)
    hi3 = lax.bitcast_convert_type(gp3 & jnp.int32(-65536), jnp.float32)
    f2 = lax.bitcast_convert_type(gt3_ref[1:2], jnp.float32)
    a0, a1, a2 = lo3, hi3, f2
    for s in range(16):
        a0 = a0 + parts_ref[3 * s:3 * s + 1]
        a1 = a1 + parts_ref[3 * s + 1:3 * s + 2]
        a2 = a2 + parts_ref[3 * s + 2:3 * s + 3]
    agg = jnp.concatenate([a0, a1, a2], axis=0)
    h3 = agg * dinv_ref[...] + b3_ref[...]
    gids = lax.broadcasted_iota(jnp.int32, (G, NP), 0)
    oh = (gids == batch_ref[...]).astype(jnp.float32)
    pooled = lax.dot_general(oh, h3, (((1,), (1,)), ((), ())),
                             preferred_element_type=jnp.float32)
    m = jnp.max(pooled, axis=1, keepdims=True)
    ex = jnp.exp(pooled - m)
    lse = jnp.log(jnp.sum(ex, axis=1, keepdims=True))
    out_ref[...] = pooled - m - lse


def _tc3(parts3, gt3, dinv, b3_col, batch2d):
    return pl.pallas_call(
        _tc3_body,
        out_shape=jax.ShapeDtypeStruct((G, P), jnp.float32),
    )(parts3, gt3, dinv, b3_col, batch2d)


def kernel(x, edge_index, batch, W1, b1, Wm1, bm1, W2, b2, Wm2, bm2, W3, b3):
    ei = edge_index.astype(jnp.int32)
    packed = ei[1] * 65536 + ei[0]

    xp = jnp.pad(x, ((0, NP - N), (0, 0)))
    batch2d = jnp.pad(batch.astype(jnp.int32), (0, NP - N),
                      constant_values=G).reshape(1, NP)
    b1c = b1.reshape(H, 1)
    bm1c = bm1.reshape(H, 1)
    b2c = b2.reshape(H, 1)
    bm2c = bm2.reshape(H, 1)
    b3c = b3.reshape(P, 1)

    deg_parts = _deg_partials(packed)
    dinv, gp1 = _tc1(deg_parts, xp, W1)

    agg1 = _edge_agg_pairs(gp1, packed)
    gp2 = _tc_mid(agg1, gp1, dinv, b1c, Wm1, bm1c, W2, H, True)

    agg2 = _edge_agg_pairs(gp2, packed)
    gt3 = _tc_mid(agg2, gp2, dinv, b2c, Wm2, bm2c, W3, P, False)

    parts3 = _edge_agg_p3(gt3, packed)
    return _tc3(parts3, gt3, dinv, b3c, batch2d)
